# baseline (speedup 1.0000x reference)
import functools

import jax
import jax.numpy as jnp
from jax.experimental import pallas as pl

EPS = 1e-5
KNN = 16
F32 = jnp.float32


HI = jax.lax.Precision.HIGHEST
BF16 = jnp.bfloat16


def _dotf(a, b):
    return jnp.dot(a, b, preferred_element_type=F32, precision=HI)


def _dotx(a, b):
    return jnp.dot(a.astype(BF16), b.astype(BF16), preferred_element_type=F32)


def _finalize_stats(st, g, be, ne):
    tot = jnp.sum(st, axis=0)
    m = tot[0:1, :] * (1.0 / ne)
    v = tot[1:2, :] * (1.0 / ne) - m * m
    scale = g * (1.0 / jnp.sqrt(v + EPS))
    shift = be - m * scale
    return scale, shift


def _pass1_kernel(x_ref, w1_ref, b1_ref, h1_ref, st_ref, *, n, h, gpp):
    w1 = w1_ref[...]
    b1 = b1_ref[...]
    col = jax.lax.broadcasted_iota(jnp.int32, (n, n), 1)
    d2s, ssums, ssqs, xtris = [], [], [], []
    for g in range(gpp):
        xg = x_ref[g]
        xsq = xg * xg
        sqr = jnp.sum(xsq, axis=1, keepdims=True)
        sqc = jnp.transpose(sqr)
        xb = xg.astype(BF16)
        gram = jax.lax.dot_general(xb, xb,
                                   (((1,), (1,)), ((), ())),
                                   preferred_element_type=F32)
        d2s.append(sqr + sqc - 2.0 * gram)
        x_hi = xb.astype(F32)
        r1 = xg - x_hi
        x_mid = r1.astype(BF16).astype(F32)
        x_lo = (r1 - x_mid).astype(BF16).astype(F32)
        xtris.append((x_hi.astype(BF16), x_mid.astype(BF16),
                      x_lo.astype(BF16)))
        ssums.append(jnp.zeros((1, h), F32))
        ssqs.append(jnp.zeros((1, h), F32))
    d = x_ref.shape[2]
    for kk in range(KNN):
        for g in range(gpp):
            xg = x_ref[g]
            d2 = d2s[g]
            mval = jnp.min(d2, axis=1, keepdims=True)
            cand = jnp.where(d2 == mval, col, n)
            sidx = jnp.min(cand, axis=1, keepdims=True)
            sel = col == sidx
            selb = sel.astype(BF16)
            xh, xm, xl = xtris[g]
            xj = (jnp.dot(selb, xh, preferred_element_type=F32) +
                  jnp.dot(selb, xm, preferred_element_type=F32) +
                  jnp.dot(selb, xl, preferred_element_type=F32))
            e = jnp.concatenate([xg, xj - xg], axis=1)
            hk = _dotx(e, w1) + b1
            h1_ref[g, kk] = hk
            ssums[g] = ssums[g] + jnp.sum(hk, axis=0, keepdims=True)
            ssqs[g] = ssqs[g] + jnp.sum(hk * hk, axis=0, keepdims=True)
            d2s[g] = jnp.where(sel, jnp.float32(jnp.inf), d2)
    for g in range(gpp):
        st_ref[g, 0:1, :] = ssums[g]
        st_ref[g, 1:2, :] = ssqs[g]


def _pass2_kernel(h1_ref, st_ref, w2_ref, b2_ref, g1_ref, be1_ref,
                  h2_ref, st2_ref, *, ne, h):
    scale, shift = _finalize_stats(st_ref[...], g1_ref[...], be1_ref[...], ne)
    w2 = w2_ref[...]
    b2 = b2_ref[...]
    ssum = jnp.zeros((1, h), F32)
    ssq = jnp.zeros((1, h), F32)
    for kk in range(KNN):
        ha = jnp.maximum(h1_ref[0, kk] * scale + shift, 0.0)
        hk = _dotx(ha, w2) + b2
        h2_ref[0, kk] = hk
        ssum = ssum + jnp.sum(hk, axis=0, keepdims=True)
        ssq = ssq + jnp.sum(hk * hk, axis=0, keepdims=True)
    st2_ref[0, 0:1, :] = ssum
    st2_ref[0, 1:2, :] = ssq


def _pass3_kernel(h2_ref, st2_ref, w3_ref, b3_ref, g2_ref, be2_ref,
                  o_ref, *, ne, n, h, per_edge):
    scale, shift = _finalize_stats(st2_ref[...], g2_ref[...], be2_ref[...], ne)
    w3 = w3_ref[...]
    if per_edge:
        acc = jnp.zeros((n, h), F32)
        for kk in range(KNN):
            ha = jnp.maximum(h2_ref[0, kk] * scale + shift, 0.0)
            acc = acc + _dotx(ha, w3)
        o_ref[0] = acc * (1.0 / KNN) + b3_ref[...]
    else:
        acc = jnp.zeros((n, h), F32)
        for kk in range(KNN):
            acc = acc + jnp.maximum(h2_ref[0, kk] * scale + shift, 0.0)
        acc = acc * (1.0 / KNN)
        o_ref[0] = _dotx(acc, w3) + b3_ref[...]


def _edge_conv(x3, p, per_edge_l3=True):
    bq, n, d = x3.shape
    hdim = p['W1'].shape[1]
    ne = float(bq * n * KNN)
    row = lambda a: a.reshape(1, -1)
    full2 = lambda arr: pl.BlockSpec(arr.shape, lambda b: (0,) * arr.ndim)

    gpp = 2
    h1, st1 = pl.pallas_call(
        functools.partial(_pass1_kernel, n=n, h=hdim, gpp=gpp),
        grid=(bq // gpp,),
        in_specs=[pl.BlockSpec((gpp, n, d), lambda b: (b, 0, 0)),
                  full2(p['W1']),
                  pl.BlockSpec((1, hdim), lambda b: (0, 0))],
        out_specs=[pl.BlockSpec((gpp, KNN, n, hdim), lambda b: (b, 0, 0, 0)),
                   pl.BlockSpec((gpp, 2, hdim), lambda b: (b, 0, 0))],
        out_shape=[jax.ShapeDtypeStruct((bq, KNN, n, hdim), F32),
                   jax.ShapeDtypeStruct((bq, 2, hdim), F32)],
    )(x3, p['W1'], row(p['b1']))

    h2, st2 = pl.pallas_call(
        functools.partial(_pass2_kernel, ne=ne, h=hdim),
        grid=(bq,),
        in_specs=[pl.BlockSpec((1, KNN, n, hdim), lambda b: (b, 0, 0, 0)),
                  pl.BlockSpec((bq, 2, hdim), lambda b: (0, 0, 0)),
                  full2(p['W2']),
                  pl.BlockSpec((1, hdim), lambda b: (0, 0)),
                  pl.BlockSpec((1, hdim), lambda b: (0, 0)),
                  pl.BlockSpec((1, hdim), lambda b: (0, 0))],
        out_specs=[pl.BlockSpec((1, KNN, n, hdim), lambda b: (b, 0, 0, 0)),
                   pl.BlockSpec((1, 2, hdim), lambda b: (b, 0, 0))],
        out_shape=[jax.ShapeDtypeStruct((bq, KNN, n, hdim), F32),
                   jax.ShapeDtypeStruct((bq, 2, hdim), F32)],
    )(h1, st1, p['W2'], row(p['b2']), row(p['g1']), row(p['be1']))

    out = pl.pallas_call(
        functools.partial(_pass3_kernel, ne=ne, n=n, h=hdim,
                          per_edge=per_edge_l3),
        grid=(bq,),
        in_specs=[pl.BlockSpec((1, KNN, n, hdim), lambda b: (b, 0, 0, 0)),
                  pl.BlockSpec((bq, 2, hdim), lambda b: (0, 0, 0)),
                  full2(p['W3']),
                  pl.BlockSpec((1, hdim), lambda b: (0, 0)),
                  pl.BlockSpec((1, hdim), lambda b: (0, 0)),
                  pl.BlockSpec((1, hdim), lambda b: (0, 0))],
        out_specs=pl.BlockSpec((1, n, hdim), lambda b: (b, 0, 0)),
        out_shape=jax.ShapeDtypeStruct((bq, n, hdim), F32),
    )(h2, st2, p['W3'], row(p['b3']), row(p['g2']), row(p['be2']))
    return out


def _bn_nodes_kernel(x_ref, g_ref, b_ref, o_ref):
    x = x_ref[...]
    m = jnp.mean(x, axis=0, keepdims=True)
    xc = x - m
    v = jnp.mean(xc * xc, axis=0, keepdims=True)
    o_ref[...] = g_ref[...] * xc * (1.0 / jnp.sqrt(v + EPS)) + b_ref[...]


def _bn_rows(x, g, be):
    m = jnp.mean(x, axis=0, keepdims=True)
    xc = x - m
    v = jnp.mean(xc * xc, axis=0, keepdims=True)
    return g * xc * (1.0 / jnp.sqrt(v + EPS)) + be


def _final_kernel(xc_ref, u_ref, gu_ref, bu_ref,
                  w1u_ref, w1x_ref, b1_ref, g1_ref, be1_ref,
                  w2_ref, b2_ref, g2_ref, be2_ref,
                  w3_ref, b3_ref, o_ref):
    xc = xc_ref[...]
    u = u_ref[...]
    u1 = _bn_rows(u, gu_ref[...], bu_ref[...])
    u2 = jnp.mean(xc, axis=1)
    hh = _dotx(u1, w1u_ref[...]) + _dotx(u2, w1x_ref[...]) + b1_ref[...]
    hh = jnp.maximum(_bn_rows(hh, g1_ref[...], be1_ref[...]), 0.0)
    hh = _dotx(hh, w2_ref[...]) + b2_ref[...]
    hh = jnp.maximum(_bn_rows(hh, g2_ref[...], be2_ref[...]), 0.0)
    o_ref[...] = _dotx(hh, w3_ref[...]) + b3_ref[...]


def kernel(x, batch, u, params):
    del batch
    nb, gd = u.shape
    nn, d = x.shape
    npg = nn // nb
    row = lambda a: a.reshape(1, -1)

    x1 = pl.pallas_call(
        _bn_nodes_kernel,
        out_shape=jax.ShapeDtypeStruct((nn, d), F32),
    )(x, row(params['bn_x']['g']), row(params['bn_x']['b']))

    c1 = _edge_conv(x.reshape(nb, npg, d), params['conv1'])
    xc = jnp.concatenate([x1, c1.reshape(nn, -1)], axis=-1)
    c2 = _edge_conv(xc.reshape(nb, npg, -1), params['conv2'])
    xc = jnp.concatenate([x1, c2.reshape(nn, -1)], axis=-1)
    c3 = _edge_conv(xc.reshape(nb, npg, -1), params['conv3'],
                    per_edge_l3=False)
    xc = jnp.concatenate([x1, c3.reshape(nn, -1)], axis=-1)

    po = params['out']
    w1u = po['W1'][:gd, :]
    w1x = po['W1'][gd:, :]
    xc3 = xc.reshape(nb, npg, -1)
    out = pl.pallas_call(
        _final_kernel,
        out_shape=jax.ShapeDtypeStruct((nb, 1), F32),
    )(xc3, u, row(params['bn_u']['g']), row(params['bn_u']['b']),
      w1u, w1x, row(po['b1']), row(po['g1']), row(po['be1']),
      po['W2'], row(po['b2']), row(po['g2']), row(po['be2']),
      po['W3'], row(po['b3']))
    return out


# speedup vs baseline: 1.0137x; 1.0137x over previous
import functools

import jax
import jax.numpy as jnp
from jax.experimental import pallas as pl

EPS = 1e-5
KNN = 16
F32 = jnp.float32


HI = jax.lax.Precision.HIGHEST
BF16 = jnp.bfloat16


def _dotf(a, b):
    return jnp.dot(a, b, preferred_element_type=F32, precision=HI)


def _dotx(a, b):
    return jnp.dot(a.astype(BF16), b.astype(BF16), preferred_element_type=F32)


def _finalize_stats(st, g, be, ne):
    tot = jnp.sum(st, axis=0)
    m = tot[0:1, :] * (1.0 / ne)
    v = tot[1:2, :] * (1.0 / ne) - m * m
    scale = g * (1.0 / jnp.sqrt(v + EPS))
    shift = be - m * scale
    return scale, shift


def _pass1_kernel(x_ref, w1_ref, b1_ref, h1_ref, st_ref, *, n, h, gpp):
    w1 = w1_ref[...]
    b1 = b1_ref[...]
    col = jax.lax.broadcasted_iota(jnp.int32, (n, n), 1)
    d2s, ssums, ssqs, xtris = [], [], [], []
    for g in range(gpp):
        xg = x_ref[g]
        xsq = xg * xg
        sqr = jnp.sum(xsq, axis=1, keepdims=True)
        sqc = jnp.transpose(sqr)
        xb = xg.astype(BF16)
        gram = jax.lax.dot_general(xb, xb,
                                   (((1,), (1,)), ((), ())),
                                   preferred_element_type=F32)
        d2s.append(sqr + sqc - 2.0 * gram)
        x_hi = xb.astype(F32)
        r1 = xg - x_hi
        x_mid = r1.astype(BF16).astype(F32)
        x_lo = (r1 - x_mid).astype(BF16).astype(F32)
        xtris.append((x_hi.astype(BF16), x_mid.astype(BF16),
                      x_lo.astype(BF16)))
        ssums.append(jnp.zeros((1, h), F32))
        ssqs.append(jnp.zeros((1, h), F32))
    d = x_ref.shape[2]
    for kk in range(KNN):
        for g in range(gpp):
            xg = x_ref[g]
            d2 = d2s[g]
            mval = jnp.min(d2, axis=1, keepdims=True)
            cand = jnp.where(d2 == mval, col, n)
            sidx = jnp.min(cand, axis=1, keepdims=True)
            sel = col == sidx
            selb = sel.astype(BF16)
            xh, xm, xl = xtris[g]
            xj = (jnp.dot(selb, xh, preferred_element_type=F32) +
                  jnp.dot(selb, xm, preferred_element_type=F32) +
                  jnp.dot(selb, xl, preferred_element_type=F32))
            e = jnp.concatenate([xg, xj - xg], axis=1)
            hk = _dotx(e, w1) + b1
            h1_ref[g, kk] = hk
            ssums[g] = ssums[g] + jnp.sum(hk, axis=0, keepdims=True)
            ssqs[g] = ssqs[g] + jnp.sum(hk * hk, axis=0, keepdims=True)
            d2s[g] = jnp.where(sel, jnp.float32(jnp.inf), d2)
    for g in range(gpp):
        st_ref[g, 0:1, :] = ssums[g]
        st_ref[g, 1:2, :] = ssqs[g]


def _pass2_kernel(h1_ref, st_ref, w2_ref, b2_ref, g1_ref, be1_ref,
                  h2_ref, st2_ref, *, ne, h):
    scale, shift = _finalize_stats(st_ref[...], g1_ref[...], be1_ref[...], ne)
    w2 = w2_ref[...]
    b2 = b2_ref[...]
    ssum = jnp.zeros((1, h), F32)
    ssq = jnp.zeros((1, h), F32)
    for kk in range(KNN):
        ha = jnp.maximum(h1_ref[0, kk] * scale + shift, 0.0)
        hk = _dotx(ha, w2) + b2
        h2_ref[0, kk] = hk
        ssum = ssum + jnp.sum(hk, axis=0, keepdims=True)
        ssq = ssq + jnp.sum(hk * hk, axis=0, keepdims=True)
    st2_ref[0, 0:1, :] = ssum
    st2_ref[0, 1:2, :] = ssq


def _pass3_kernel(h2_ref, st2_ref, w3_ref, b3_ref, g2_ref, be2_ref,
                  o_ref, *, ne, n, h, per_edge):
    scale, shift = _finalize_stats(st2_ref[...], g2_ref[...], be2_ref[...], ne)
    w3 = w3_ref[...]
    if per_edge:
        acc = jnp.zeros((n, h), F32)
        for kk in range(KNN):
            ha = jnp.maximum(h2_ref[0, kk] * scale + shift, 0.0)
            acc = acc + _dotx(ha, w3)
        o_ref[0] = acc * (1.0 / KNN) + b3_ref[...]
    else:
        acc = jnp.zeros((n, h), F32)
        for kk in range(KNN):
            acc = acc + jnp.maximum(h2_ref[0, kk] * scale + shift, 0.0)
        acc = acc * (1.0 / KNN)
        o_ref[0] = _dotx(acc, w3) + b3_ref[...]


def _edge_conv(x3, p, per_edge_l3=True):
    bq, n, d = x3.shape
    hdim = p['W1'].shape[1]
    ne = float(bq * n * KNN)
    row = lambda a: a.reshape(1, -1)
    full2 = lambda arr: pl.BlockSpec(arr.shape, lambda b: (0,) * arr.ndim)

    gpp = 5
    h1, st1 = pl.pallas_call(
        functools.partial(_pass1_kernel, n=n, h=hdim, gpp=gpp),
        grid=(bq // gpp,),
        in_specs=[pl.BlockSpec((gpp, n, d), lambda b: (b, 0, 0)),
                  full2(p['W1']),
                  pl.BlockSpec((1, hdim), lambda b: (0, 0))],
        out_specs=[pl.BlockSpec((gpp, KNN, n, hdim), lambda b: (b, 0, 0, 0)),
                   pl.BlockSpec((gpp, 2, hdim), lambda b: (b, 0, 0))],
        out_shape=[jax.ShapeDtypeStruct((bq, KNN, n, hdim), F32),
                   jax.ShapeDtypeStruct((bq, 2, hdim), F32)],
    )(x3, p['W1'], row(p['b1']))

    h2, st2 = pl.pallas_call(
        functools.partial(_pass2_kernel, ne=ne, h=hdim),
        grid=(bq,),
        in_specs=[pl.BlockSpec((1, KNN, n, hdim), lambda b: (b, 0, 0, 0)),
                  pl.BlockSpec((bq, 2, hdim), lambda b: (0, 0, 0)),
                  full2(p['W2']),
                  pl.BlockSpec((1, hdim), lambda b: (0, 0)),
                  pl.BlockSpec((1, hdim), lambda b: (0, 0)),
                  pl.BlockSpec((1, hdim), lambda b: (0, 0))],
        out_specs=[pl.BlockSpec((1, KNN, n, hdim), lambda b: (b, 0, 0, 0)),
                   pl.BlockSpec((1, 2, hdim), lambda b: (b, 0, 0))],
        out_shape=[jax.ShapeDtypeStruct((bq, KNN, n, hdim), F32),
                   jax.ShapeDtypeStruct((bq, 2, hdim), F32)],
    )(h1, st1, p['W2'], row(p['b2']), row(p['g1']), row(p['be1']))

    out = pl.pallas_call(
        functools.partial(_pass3_kernel, ne=ne, n=n, h=hdim,
                          per_edge=per_edge_l3),
        grid=(bq,),
        in_specs=[pl.BlockSpec((1, KNN, n, hdim), lambda b: (b, 0, 0, 0)),
                  pl.BlockSpec((bq, 2, hdim), lambda b: (0, 0, 0)),
                  full2(p['W3']),
                  pl.BlockSpec((1, hdim), lambda b: (0, 0)),
                  pl.BlockSpec((1, hdim), lambda b: (0, 0)),
                  pl.BlockSpec((1, hdim), lambda b: (0, 0))],
        out_specs=pl.BlockSpec((1, n, hdim), lambda b: (b, 0, 0)),
        out_shape=jax.ShapeDtypeStruct((bq, n, hdim), F32),
    )(h2, st2, p['W3'], row(p['b3']), row(p['g2']), row(p['be2']))
    return out


def _bn_nodes_kernel(x_ref, g_ref, b_ref, o_ref):
    x = x_ref[...]
    m = jnp.mean(x, axis=0, keepdims=True)
    xc = x - m
    v = jnp.mean(xc * xc, axis=0, keepdims=True)
    o_ref[...] = g_ref[...] * xc * (1.0 / jnp.sqrt(v + EPS)) + b_ref[...]


def _bn_rows(x, g, be):
    m = jnp.mean(x, axis=0, keepdims=True)
    xc = x - m
    v = jnp.mean(xc * xc, axis=0, keepdims=True)
    return g * xc * (1.0 / jnp.sqrt(v + EPS)) + be


def _final_kernel(xc_ref, u_ref, gu_ref, bu_ref,
                  w1u_ref, w1x_ref, b1_ref, g1_ref, be1_ref,
                  w2_ref, b2_ref, g2_ref, be2_ref,
                  w3_ref, b3_ref, o_ref):
    xc = xc_ref[...]
    u = u_ref[...]
    u1 = _bn_rows(u, gu_ref[...], bu_ref[...])
    u2 = jnp.mean(xc, axis=1)
    hh = _dotx(u1, w1u_ref[...]) + _dotx(u2, w1x_ref[...]) + b1_ref[...]
    hh = jnp.maximum(_bn_rows(hh, g1_ref[...], be1_ref[...]), 0.0)
    hh = _dotx(hh, w2_ref[...]) + b2_ref[...]
    hh = jnp.maximum(_bn_rows(hh, g2_ref[...], be2_ref[...]), 0.0)
    o_ref[...] = _dotx(hh, w3_ref[...]) + b3_ref[...]


def kernel(x, batch, u, params):
    del batch
    nb, gd = u.shape
    nn, d = x.shape
    npg = nn // nb
    row = lambda a: a.reshape(1, -1)

    x1 = pl.pallas_call(
        _bn_nodes_kernel,
        out_shape=jax.ShapeDtypeStruct((nn, d), F32),
    )(x, row(params['bn_x']['g']), row(params['bn_x']['b']))

    c1 = _edge_conv(x.reshape(nb, npg, d), params['conv1'])
    xc = jnp.concatenate([x1, c1.reshape(nn, -1)], axis=-1)
    c2 = _edge_conv(xc.reshape(nb, npg, -1), params['conv2'])
    xc = jnp.concatenate([x1, c2.reshape(nn, -1)], axis=-1)
    c3 = _edge_conv(xc.reshape(nb, npg, -1), params['conv3'],
                    per_edge_l3=False)
    xc = jnp.concatenate([x1, c3.reshape(nn, -1)], axis=-1)

    po = params['out']
    w1u = po['W1'][:gd, :]
    w1x = po['W1'][gd:, :]
    xc3 = xc.reshape(nb, npg, -1)
    out = pl.pallas_call(
        _final_kernel,
        out_shape=jax.ShapeDtypeStruct((nb, 1), F32),
    )(xc3, u, row(params['bn_u']['g']), row(params['bn_u']['b']),
      w1u, w1x, row(po['b1']), row(po['g1']), row(po['be1']),
      po['W2'], row(po['b2']), row(po['g2']), row(po['be2']),
      po['W3'], row(po['b3']))
    return out
